# baseline (speedup 1.0000x reference)
import functools

import jax
import jax.numpy as jnp
from jax.experimental import pallas as pl
from jax.experimental.pallas import tpu as pltpu

CAP_FACTOR = 1.25


def _moe_block_kernel(x_ref, ws_ref, bs_ref, w1_ref, b1_ref, w2_ref, b2_ref,
                      out_ref, counts_ref, *, capacity, blk, n_experts):
    i = pl.program_id(0)

    @pl.when(i == 0)
    def _init():
        counts_ref[...] = jnp.zeros_like(counts_ref)

    x = x_ref[...]

    logits = jnp.dot(x, ws_ref[...], preferred_element_type=jnp.float32)
    logits = logits + bs_ref[...]
    m = jnp.max(logits, axis=-1, keepdims=True)
    e = jnp.exp(logits - m)
    probs = e / jnp.sum(e, axis=-1, keepdims=True)
    p_max = jnp.max(probs, axis=-1, keepdims=True)
    col = jax.lax.broadcasted_iota(jnp.int32, (blk, n_experts), 1)
    routes = jnp.min(jnp.where(probs == p_max, col, n_experts), axis=-1,
                     keepdims=True)
    onehot = (routes == col).astype(jnp.float32)

    r = jax.lax.broadcasted_iota(jnp.int32, (blk, blk), 0)
    c = jax.lax.broadcasted_iota(jnp.int32, (blk, blk), 1)
    tri = (r >= c).astype(jnp.float32)
    csum = jnp.dot(tri, onehot, preferred_element_type=jnp.float32)
    base = counts_ref[...]
    pos = (jnp.sum(csum * onehot, axis=-1, keepdims=True) - 1.0
           + jnp.sum(onehot * base, axis=-1, keepdims=True))
    counts_ref[...] = base + jnp.sum(onehot, axis=0, keepdims=True)
    kept = pos < capacity

    ff = w1_ref.shape[1]
    ffc = 768
    y = b2_ref[...] * jnp.ones((x.shape[0], 1), jnp.float32)
    for c in range(ff // ffc):
        hc = jnp.dot(x, w1_ref[:, c * ffc:(c + 1) * ffc],
                     preferred_element_type=jnp.float32)
        hc = hc + b1_ref[:, c * ffc:(c + 1) * ffc]
        hc = 0.5 * hc * (1.0 + jax.lax.erf(hc * 0.7071067811865476))
        y = y + jnp.dot(hc, w2_ref[c * ffc:(c + 1) * ffc, :],
                        preferred_element_type=jnp.float32)

    out_ref[...] = jnp.where(kept, y * p_max, x)


def kernel(x, W_switch, b_switch, W1, b1, W2, b2):
    T, D = x.shape
    E = W_switch.shape[1]
    FF = W1.shape[1]
    capacity = int(CAP_FACTOR * T / E)
    blk = 1024
    grid = T // blk

    body = functools.partial(_moe_block_kernel, capacity=capacity, blk=blk,
                             n_experts=E)
    return pl.pallas_call(
        body,
        grid=(grid,),
        in_specs=[
            pl.BlockSpec((blk, D), lambda i: (i, 0)),
            pl.BlockSpec((D, E), lambda i: (0, 0)),
            pl.BlockSpec((1, E), lambda i: (0, 0)),
            pl.BlockSpec((D, FF), lambda i: (0, 0)),
            pl.BlockSpec((1, FF), lambda i: (0, 0)),
            pl.BlockSpec((FF, D), lambda i: (0, 0)),
            pl.BlockSpec((1, D), lambda i: (0, 0)),
        ],
        out_specs=pl.BlockSpec((blk, D), lambda i: (i, 0)),
        out_shape=jax.ShapeDtypeStruct((T, D), x.dtype),
        scratch_shapes=[pltpu.VMEM((1, E), jnp.float32)],
    )(x, W_switch, b_switch.reshape(1, E),
      W1, b1.reshape(1, FF),
      W2, b2.reshape(1, D))


# speedup vs baseline: 1.0076x; 1.0076x over previous
import functools

import jax
import jax.numpy as jnp
from jax.experimental import pallas as pl
from jax.experimental.pallas import tpu as pltpu

CAP_FACTOR = 1.25


def _moe_block_kernel(x_ref, ws_ref, bs_ref, w1_ref, b1_ref, w2_ref, b2_ref,
                      out_ref, counts_ref, *, capacity, blk, n_experts):
    i = pl.program_id(0)

    @pl.when(i == 0)
    def _init():
        counts_ref[...] = jnp.zeros_like(counts_ref)

    x = x_ref[...]

    logits = jnp.dot(x, ws_ref[...], preferred_element_type=jnp.float32)
    logits = logits + bs_ref[...]
    m = jnp.max(logits, axis=-1, keepdims=True)
    e = jnp.exp(logits - m)
    probs = e / jnp.sum(e, axis=-1, keepdims=True)
    p_max = jnp.max(probs, axis=-1, keepdims=True)
    col = jax.lax.broadcasted_iota(jnp.int32, (blk, n_experts), 1)
    routes = jnp.min(jnp.where(probs == p_max, col, n_experts), axis=-1,
                     keepdims=True)
    onehot = (routes == col).astype(jnp.float32)

    sub = min(blk, 512)
    r = jax.lax.broadcasted_iota(jnp.int32, (sub, sub), 0)
    c = jax.lax.broadcasted_iota(jnp.int32, (sub, sub), 1)
    tri = (r >= c).astype(jnp.float32)
    base = counts_ref[...]
    pos_parts = []
    off = base
    for k in range(blk // sub):
        oh_k = onehot[k * sub:(k + 1) * sub, :]
        csum_k = jnp.dot(tri, oh_k, preferred_element_type=jnp.float32)
        pos_k = (jnp.sum(csum_k * oh_k, axis=-1, keepdims=True) - 1.0
                 + jnp.sum(oh_k * off, axis=-1, keepdims=True))
        pos_parts.append(pos_k)
        off = off + jnp.sum(oh_k, axis=0, keepdims=True)
    counts_ref[...] = off
    pos = jnp.concatenate(pos_parts, axis=0)
    kept = pos < capacity

    ff = w1_ref.shape[1]
    ffc = 768
    y = b2_ref[...] * jnp.ones((x.shape[0], 1), jnp.float32)
    for c in range(ff // ffc):
        hc = jnp.dot(x, w1_ref[:, c * ffc:(c + 1) * ffc],
                     preferred_element_type=jnp.float32)
        hc = hc + b1_ref[:, c * ffc:(c + 1) * ffc]
        hc = 0.5 * hc * (1.0 + jax.lax.erf(hc * 0.7071067811865476))
        y = y + jnp.dot(hc, w2_ref[c * ffc:(c + 1) * ffc, :],
                        preferred_element_type=jnp.float32)

    out_ref[...] = jnp.where(kept, y * p_max, x)


def kernel(x, W_switch, b_switch, W1, b1, W2, b2):
    T, D = x.shape
    E = W_switch.shape[1]
    FF = W1.shape[1]
    capacity = int(CAP_FACTOR * T / E)
    blk = min(2048, T)
    grid = T // blk

    body = functools.partial(_moe_block_kernel, capacity=capacity, blk=blk,
                             n_experts=E)
    return pl.pallas_call(
        body,
        grid=(grid,),
        in_specs=[
            pl.BlockSpec((blk, D), lambda i: (i, 0)),
            pl.BlockSpec((D, E), lambda i: (0, 0)),
            pl.BlockSpec((1, E), lambda i: (0, 0)),
            pl.BlockSpec((D, FF), lambda i: (0, 0)),
            pl.BlockSpec((1, FF), lambda i: (0, 0)),
            pl.BlockSpec((FF, D), lambda i: (0, 0)),
            pl.BlockSpec((1, D), lambda i: (0, 0)),
        ],
        out_specs=pl.BlockSpec((blk, D), lambda i: (i, 0)),
        out_shape=jax.ShapeDtypeStruct((T, D), x.dtype),
        scratch_shapes=[pltpu.VMEM((1, E), jnp.float32)],
    )(x, W_switch, b_switch.reshape(1, E),
      W1, b1.reshape(1, FF),
      W2, b2.reshape(1, D))


# speedup vs baseline: 1.1068x; 1.0984x over previous
import functools

import jax
import jax.numpy as jnp
from jax.experimental import pallas as pl
from jax.experimental.pallas import tpu as pltpu

CAP_FACTOR = 1.25


def _moe_block_kernel(x_ref, ws_ref, bs_ref, w1_ref, b1_ref, w2_ref, b2_ref,
                      out_ref, counts_ref, *, capacity, blk, n_experts):
    i = pl.program_id(0)

    @pl.when(i == 0)
    def _init():
        counts_ref[...] = jnp.zeros_like(counts_ref)

    x = x_ref[...]

    h = jnp.dot(x, w1_ref[...], preferred_element_type=jnp.float32)
    h = h + b1_ref[...]
    h = 0.5 * h * (1.0 + jax.lax.erf(h * 0.7071067811865476))

    logits = jnp.dot(x, ws_ref[...], preferred_element_type=jnp.float32)
    logits = logits + bs_ref[...]
    m = jnp.max(logits, axis=-1, keepdims=True)
    e = jnp.exp(logits - m)
    probs = e / jnp.sum(e, axis=-1, keepdims=True)
    p_max = jnp.max(probs, axis=-1, keepdims=True)
    col = jax.lax.broadcasted_iota(jnp.int32, (blk, n_experts), 1)
    routes = jnp.min(jnp.where(probs == p_max, col, n_experts), axis=-1,
                     keepdims=True)
    onehot = (routes == col).astype(jnp.float32)

    r = jax.lax.broadcasted_iota(jnp.int32, (blk, blk), 0)
    c = jax.lax.broadcasted_iota(jnp.int32, (blk, blk), 1)
    tri = (r >= c).astype(jnp.float32)
    csum = jnp.dot(tri, onehot, preferred_element_type=jnp.float32)
    base = counts_ref[...]
    pos = (jnp.sum(csum * onehot, axis=-1, keepdims=True) - 1.0
           + jnp.sum(onehot * base, axis=-1, keepdims=True))
    counts_ref[...] = base + jnp.sum(onehot, axis=0, keepdims=True)
    kept = pos < capacity

    y = jnp.dot(h, w2_ref[...], preferred_element_type=jnp.float32)
    y = y + b2_ref[...]

    out_ref[...] = jnp.where(kept, y * p_max, x)


def kernel(x, W_switch, b_switch, W1, b1, W2, b2):
    T, D = x.shape
    E = W_switch.shape[1]
    FF = W1.shape[1]
    capacity = int(CAP_FACTOR * T / E)
    blk = min(1024, T)
    grid = T // blk

    body = functools.partial(_moe_block_kernel, capacity=capacity, blk=blk,
                             n_experts=E)
    return pl.pallas_call(
        body,
        grid=(grid,),
        in_specs=[
            pl.BlockSpec((blk, D), lambda i: (i, 0)),
            pl.BlockSpec((D, E), lambda i: (0, 0)),
            pl.BlockSpec((1, E), lambda i: (0, 0)),
            pl.BlockSpec((D, FF), lambda i: (0, 0)),
            pl.BlockSpec((1, FF), lambda i: (0, 0)),
            pl.BlockSpec((FF, D), lambda i: (0, 0)),
            pl.BlockSpec((1, D), lambda i: (0, 0)),
        ],
        out_specs=pl.BlockSpec((blk, D), lambda i: (i, 0)),
        out_shape=jax.ShapeDtypeStruct((T, D), x.dtype),
        scratch_shapes=[pltpu.VMEM((1, E), jnp.float32)],
    )(x, W_switch, b_switch.reshape(1, E),
      W1, b1.reshape(1, FF),
      W2, b2.reshape(1, D))
